# baseline (speedup 1.0000x reference)
import jax
import jax.numpy as jnp
from jax.experimental import pallas as pl
from jax.experimental.pallas import tpu as pltpu


_BLOCK = 2_560_000


def _body(x_ref, o_ref):
    o_ref[...] = x_ref[...]

    @pl.when(pl.program_id(0) == 0)
    def _():
        o_ref[0:128] = jnp.ones((128,), o_ref.dtype)


def kernel(x):
    n, d = x.shape
    xf = x.reshape(n * d)
    out = pl.pallas_call(
        _body,
        grid=(xf.shape[0] // _BLOCK,),
        in_specs=[pl.BlockSpec((_BLOCK,), lambda i: (i,))],
        out_specs=pl.BlockSpec((_BLOCK,), lambda i: (i,)),
        out_shape=jax.ShapeDtypeStruct(xf.shape, x.dtype),
        compiler_params=pltpu.CompilerParams(
            dimension_semantics=("arbitrary",),
        ),
    )(xf)
    return out.reshape(n, d)


# speedup vs baseline: 1.3642x; 1.3642x over previous
import jax
import jax.numpy as jnp
from jax.experimental import pallas as pl
from jax.experimental.pallas import tpu as pltpu


_BLOCK = 20000


def _body(x_ref, o_ref):
    o_ref[...] = x_ref[...]

    @pl.when(pl.program_id(0) == 0)
    def _():
        o_ref[0:2, :] = jnp.ones((2, o_ref.shape[1]), o_ref.dtype)


def kernel(x):
    n, d = x.shape
    return pl.pallas_call(
        _body,
        grid=(n // _BLOCK,),
        in_specs=[pl.BlockSpec((_BLOCK, d), lambda i: (i, 0))],
        out_specs=pl.BlockSpec((_BLOCK, d), lambda i: (i, 0)),
        out_shape=jax.ShapeDtypeStruct((n, d), x.dtype),
        compiler_params=pltpu.CompilerParams(
            dimension_semantics=("arbitrary",),
        ),
    )(x)
